# Initial kernel scaffold; baseline (speedup 1.0000x reference)
import jax
import jax.numpy as jnp
from jax.experimental import pallas as pl


def kernel(x, edge_index, W1, al1, ar1, W2, al2, ar2):
    raise NotImplementedError("write your pallas kernel here")


# speedup vs baseline: 7.6358x; 7.6358x over previous
import jax
import jax.numpy as jnp
from jax import lax
from jax.experimental import pallas as pl
from jax.experimental.pallas import tpu as pltpu
from jax.experimental.pallas import tpu_sc as plsc

_N = 10000
_E = 320000
_IN = 128
_D = 64
_H1 = 8
_NC = 41
_NEG = 0.2
_DA = 128
_NCORE = 2
_NSUB = 16
_NW = _NCORE * _NSUB
_NP = 10240
_ER = 2560
_RW = _ER // _NW
_CR = 4
_NCH = _RW // _CR
_B = 32
_NBC = _CR * 128 // _B


def _prep1_body(x_r, w_r, al_r, ar_r, haug_r, el_r, er_r, gm_r):
    h = jnp.dot(x_r[...], w_r[...], preferred_element_type=jnp.float32)
    blk = h.shape[0]
    pad = jnp.concatenate(
        [jnp.ones((blk, 1), jnp.float32),
         jnp.zeros((blk, _DA - _D - 1), jnp.float32)], axis=1)
    al = al_r[...]
    ar = ar_r[...]
    mx = []
    for hd in range(_H1):
        hh = h[:, hd * _D:(hd + 1) * _D]
        haug_r[hd, :, 0:_D] = hh
        haug_r[hd, :, _D:_DA] = pad
        el = jnp.dot(hh, al[hd], preferred_element_type=jnp.float32)
        er = jnp.dot(hh, ar[hd], preferred_element_type=jnp.float32)
        el_r[hd] = el.reshape(-1, 128)
        er_r[hd] = er.reshape(-1, 128)
        mx.append(jnp.max(el))
    bm = jnp.broadcast_to(jnp.stack(mx)[:, None], (_H1, 128))

    @pl.when(pl.program_id(0) == 0)
    def _():
        gm_r[...] = bm

    @pl.when(pl.program_id(0) > 0)
    def _():
        gm_r[...] = jnp.maximum(gm_r[...], bm)


def _prep1(x, W1, al1, ar1):
    blk = 2048
    grid = (_NP // blk,)
    return pl.pallas_call(
        _prep1_body,
        grid=grid,
        in_specs=[
            pl.BlockSpec((blk, _IN), lambda i: (i, 0)),
            pl.BlockSpec((_IN, _H1 * _D), lambda i: (0, 0)),
            pl.BlockSpec((_H1, _D), lambda i: (0, 0)),
            pl.BlockSpec((_H1, _D), lambda i: (0, 0)),
        ],
        out_specs=[
            pl.BlockSpec((_H1, blk, _DA), lambda i: (0, i, 0)),
            pl.BlockSpec((_H1, blk // 128, 128), lambda i: (0, i, 0)),
            pl.BlockSpec((_H1, blk // 128, 128), lambda i: (0, i, 0)),
            pl.BlockSpec((_H1, 128), lambda i: (0, 0)),
        ],
        out_shape=[
            jax.ShapeDtypeStruct((_H1, _NP, _DA), jnp.float32),
            jax.ShapeDtypeStruct((_H1, _NP // 128, 128), jnp.float32),
            jax.ShapeDtypeStruct((_H1, _NP // 128, 128), jnp.float32),
            jax.ShapeDtypeStruct((_H1, 128), jnp.float32),
        ],
    )(x, W1, al1, ar1)


def _prep2_body(acc_r, w_r, al_r, ar_r, haug_r, el_r, er_r, gm_r):
    a = acc_r[...]
    num = a[0, :, :, 0:_D] + a[1, :, :, 0:_D]
    s = a[0, :, :, _D:_D + 1] + a[1, :, :, _D:_D + 1]
    o = num / (s + 1e-9)
    x2 = jnp.where(o > 0, o, jnp.exp(o) - 1.0)
    blk = x2.shape[1]
    w = w_r[...]
    h2 = jnp.zeros((blk, _D), jnp.float32)
    for hd in range(_H1):
        h2 = h2 + jnp.dot(x2[hd], w[hd * _D:(hd + 1) * _D, :],
                          preferred_element_type=jnp.float32)
    pad = jnp.concatenate(
        [jnp.ones((blk, 1), jnp.float32),
         jnp.zeros((blk, _DA - _D - 1), jnp.float32)], axis=1)
    haug_r[0, :, 0:_D] = h2
    haug_r[0, :, _D:_DA] = pad
    el = jnp.dot(h2, al_r[...][0], preferred_element_type=jnp.float32)
    er = jnp.dot(h2, ar_r[...][0], preferred_element_type=jnp.float32)
    el_r[0] = el.reshape(-1, 128)
    er_r[0] = er.reshape(-1, 128)
    bm = jnp.broadcast_to(jnp.max(el), (1, 128))

    @pl.when(pl.program_id(0) == 0)
    def _():
        gm_r[...] = bm

    @pl.when(pl.program_id(0) > 0)
    def _():
        gm_r[...] = jnp.maximum(gm_r[...], bm)


def _prep2(acc1, W2p, al2p, ar2p):
    blk = 2048
    grid = (_NP // blk,)
    return pl.pallas_call(
        _prep2_body,
        grid=grid,
        in_specs=[
            pl.BlockSpec((_NCORE, _H1, blk, _DA), lambda i: (0, 0, i, 0)),
            pl.BlockSpec((_H1 * _D, _D), lambda i: (0, 0)),
            pl.BlockSpec((1, _D), lambda i: (0, 0)),
            pl.BlockSpec((1, _D), lambda i: (0, 0)),
        ],
        out_specs=[
            pl.BlockSpec((1, blk, _DA), lambda i: (0, i, 0)),
            pl.BlockSpec((1, blk // 128, 128), lambda i: (0, i, 0)),
            pl.BlockSpec((1, blk // 128, 128), lambda i: (0, i, 0)),
            pl.BlockSpec((1, 128), lambda i: (0, 0)),
        ],
        out_shape=[
            jax.ShapeDtypeStruct((1, _NP, _DA), jnp.float32),
            jax.ShapeDtypeStruct((1, _NP // 128, 128), jnp.float32),
            jax.ShapeDtypeStruct((1, _NP // 128, 128), jnp.float32),
            jax.ShapeDtypeStruct((1, 128), jnp.float32),
        ],
    )(acc1, W2p, al2p, ar2p)


def _final_body(acc_r, out_r):
    a = acc_r[...]
    num = a[0, 0, :, 0:_NC] + a[1, 0, :, 0:_NC]
    s = a[0, 0, :, _D:_D + 1] + a[1, 0, :, _D:_D + 1]
    out_r[...] = num / (s + 1e-9)


def _final(acc2):
    blk = 2000
    grid = (_N // blk,)
    return pl.pallas_call(
        _final_body,
        grid=grid,
        in_specs=[pl.BlockSpec((_NCORE, 1, blk, _DA), lambda i: (0, 0, i, 0))],
        out_specs=[pl.BlockSpec((blk, _NC), lambda i: (i, 0))],
        out_shape=[jax.ShapeDtypeStruct((_N, _NC), jnp.float32)],
    )(acc2)[0]


def _sc_edge(H):
    mesh = plsc.VectorSubcoreMesh(core_axis_name="c", subcore_axis_name="s")
    NZ = _N // _B

    def body(haug, elT, erT, gmax, esrc, edst, acc_out,
             ebsrc, ebdst, el_v, er_v, gm_v,
             exbuf, srcbuf, dstbuf, gbuf, acc_sh):
        c = lax.axis_index("c")
        s = lax.axis_index("s")
        wid = s * _NCORE + c
        pltpu.sync_copy(gmax, gm_v)

        def head(hd, _):
            pltpu.sync_copy(elT.at[hd], el_v)
            pltpu.sync_copy(erT.at[hd], er_v)
            gmv = gm_v[hd, pl.ds(0, 16)]

            def zrow(i, _):
                for kk in range(_DA // 16):
                    gbuf[i, pl.ds(kk * 16, 16)] = jnp.zeros((16,), jnp.float32)
                return 0
            lax.fori_loop(0, _B, zrow, 0)
            for z in range(20):
                zi = s * 20 + z

                @pl.when(zi < NZ)
                def _():
                    pltpu.sync_copy(gbuf, acc_sh.at[pl.ds(zi * _B, _B)])

            @pl.when(s == 0)
            def _():
                pltpu.sync_copy(gbuf.at[pl.ds(0, 16)],
                                acc_sh.at[pl.ds(NZ * _B, 16)])
            plsc.subcore_barrier()

            def chunk(ci, _):
                row0 = wid * _RW + ci * _CR
                pltpu.sync_copy(esrc.at[pl.ds(row0, _CR)], ebsrc)
                pltpu.sync_copy(edst.at[pl.ds(row0, _CR)], ebdst)
                for bi in range(_NBC):
                    r0 = bi * _B // 128
                    c0 = (bi * _B) % 128
                    for j in range(_B // 16):
                        rr = r0 + (c0 + j * 16) // 128
                        cc = (c0 + j * 16) % 128
                        src_v = ebsrc[rr, pl.ds(cc, 16)]
                        dst_v = ebdst[rr, pl.ds(cc, 16)]
                        a = plsc.load_gather(
                            el_v, [src_v >> 7, src_v & 127])
                        r = plsc.load_gather(
                            er_v, [dst_v >> 7, dst_v & 127])
                        z = a + r
                        e = jnp.where(z > 0, z, _NEG * z)
                        mb = gmv + r
                        m = jnp.where(mb > 0, mb, _NEG * mb)
                        ei = ((row0 + rr) * 128 + cc
                              + lax.iota(jnp.int32, 16))
                        ex = jnp.where(ei < _E, jnp.exp(e - m), 0.0)
                        exbuf[pl.ds(j * 16, 16)] = ex
                        srcbuf[pl.ds(j * 16, 16)] = src_v + hd * _NP
                        dstbuf[pl.ds(j * 16, 16)] = dst_v
                    pltpu.sync_copy(haug.at[srcbuf], gbuf)

                    def scale(e_i, _):
                        exv = plsc.load_gather(
                            exbuf, [jnp.zeros((16,), jnp.int32) + e_i])
                        for kk in range(5):
                            gbuf[e_i, pl.ds(kk * 16, 16)] = (
                                gbuf[e_i, pl.ds(kk * 16, 16)] * exv)
                        return 0
                    lax.fori_loop(0, _B, scale, 0)
                    pltpu.sync_copy(gbuf, acc_sh.at[dstbuf], add=True)
                return 0
            lax.fori_loop(0, _NCH, chunk, 0)
            plsc.subcore_barrier()

            @pl.when(s < 10)
            def _():
                pltpu.sync_copy(acc_sh.at[pl.ds(s * 1000, 1000)],
                                acc_out.at[c, hd, pl.ds(s * 1000, 1000)])
            plsc.subcore_barrier()
            return 0
        lax.fori_loop(0, H, head, 0)

    kern = pl.kernel(
        body,
        out_type=jax.ShapeDtypeStruct((_NCORE, H, _N, _DA), jnp.float32),
        mesh=mesh,
        compiler_params=pltpu.CompilerParams(needs_layout_passes=False),
        scratch_types=[
            pltpu.VMEM((_CR, 128), jnp.int32),
            pltpu.VMEM((_CR, 128), jnp.int32),
            pltpu.VMEM((_NP // 128, 128), jnp.float32),
            pltpu.VMEM((_NP // 128, 128), jnp.float32),
            pltpu.VMEM((H, 128), jnp.float32),
            pltpu.VMEM((_B,), jnp.float32),
            pltpu.VMEM((_B,), jnp.int32),
            pltpu.VMEM((_B,), jnp.int32),
            pltpu.VMEM((_B, _DA), jnp.float32),
            pltpu.MemorySpace.VMEM_SHARED((_N + 16, _DA), jnp.float32),
        ],
    )
    return kern


_sc_edge8 = _sc_edge(_H1)
_sc_edge1 = _sc_edge(1)


def kernel(x, edge_index, W1, al1, ar1, W2, al2, ar2):
    epad = jnp.pad(edge_index, ((0, 0), (0, _ER * 128 - _E)))
    esrc = epad[0].reshape(_ER, 128)
    edst = epad[1].reshape(_ER, 128)

    xp = jnp.pad(x, ((0, _NP - _N), (0, 0)))
    haug1, el1, er1, gm1 = _prep1(xp, W1, al1, ar1)
    acc1 = _sc_edge8(haug1.reshape(_H1 * _NP, _DA), el1, er1, gm1, esrc, edst)

    W2p = jnp.zeros((_H1 * _D, _D), jnp.float32).at[:, :_NC].set(W2)
    al2p = jnp.zeros((1, _D), jnp.float32).at[:, :_NC].set(al2)
    ar2p = jnp.zeros((1, _D), jnp.float32).at[:, :_NC].set(ar2)

    acc1p = jnp.pad(acc1, ((0, 0), (0, 0), (0, _NP - _N), (0, 0)))
    haug2, el2, er2, gm2 = _prep2(acc1p, W2p, al2p, ar2p)
    acc2 = _sc_edge1(haug2.reshape(_NP, _DA), el2, er2, gm2, esrc, edst)

    return _final(acc2)


# speedup vs baseline: 8.5973x; 1.1259x over previous
import jax
import jax.numpy as jnp
from jax import lax
from jax.experimental import pallas as pl
from jax.experimental.pallas import tpu as pltpu
from jax.experimental.pallas import tpu_sc as plsc

_N = 10000
_E = 320000
_IN = 128
_D = 64
_H1 = 8
_NC = 41
_NEG = 0.2
_DA = 128
_NCORE = 2
_NSUB = 16
_NW = _NCORE * _NSUB
_NP = 10240
_ER = 2560
_RW = _ER // _NW
_CR = 8
_NCH = _RW // _CR
_B = 64
_NBC = _CR * 128 // _B


def _prep1_body(x_r, w_r, al_r, ar_r, haug_r, el_r, er_r, gm_r):
    h = jnp.dot(x_r[...], w_r[...], preferred_element_type=jnp.float32)
    blk = h.shape[0]
    pad = jnp.concatenate(
        [jnp.ones((blk, 1), jnp.float32),
         jnp.zeros((blk, _DA - _D - 1), jnp.float32)], axis=1)
    al = al_r[...]
    ar = ar_r[...]
    mx = []
    for hd in range(_H1):
        hh = h[:, hd * _D:(hd + 1) * _D]
        haug_r[hd, :, 0:_D] = hh
        haug_r[hd, :, _D:_DA] = pad
        el = jnp.dot(hh, al[hd], preferred_element_type=jnp.float32)
        er = jnp.dot(hh, ar[hd], preferred_element_type=jnp.float32)
        el_r[hd] = el.reshape(-1, 128)
        er_r[hd] = er.reshape(-1, 128)
        mx.append(jnp.max(el))
    bm = jnp.broadcast_to(jnp.stack(mx)[:, None], (_H1, 128))

    @pl.when(pl.program_id(0) == 0)
    def _():
        gm_r[...] = bm

    @pl.when(pl.program_id(0) > 0)
    def _():
        gm_r[...] = jnp.maximum(gm_r[...], bm)


def _prep1(x, W1, al1, ar1):
    blk = 2048
    grid = (_NP // blk,)
    return pl.pallas_call(
        _prep1_body,
        grid=grid,
        in_specs=[
            pl.BlockSpec((blk, _IN), lambda i: (i, 0)),
            pl.BlockSpec((_IN, _H1 * _D), lambda i: (0, 0)),
            pl.BlockSpec((_H1, _D), lambda i: (0, 0)),
            pl.BlockSpec((_H1, _D), lambda i: (0, 0)),
        ],
        out_specs=[
            pl.BlockSpec((_H1, blk, _DA), lambda i: (0, i, 0)),
            pl.BlockSpec((_H1, blk // 128, 128), lambda i: (0, i, 0)),
            pl.BlockSpec((_H1, blk // 128, 128), lambda i: (0, i, 0)),
            pl.BlockSpec((_H1, 128), lambda i: (0, 0)),
        ],
        out_shape=[
            jax.ShapeDtypeStruct((_H1, _NP, _DA), jnp.float32),
            jax.ShapeDtypeStruct((_H1, _NP // 128, 128), jnp.float32),
            jax.ShapeDtypeStruct((_H1, _NP // 128, 128), jnp.float32),
            jax.ShapeDtypeStruct((_H1, 128), jnp.float32),
        ],
    )(x, W1, al1, ar1)


def _prep2_body(acc_r, w_r, al_r, ar_r, haug_r, el_r, er_r, gm_r):
    a = acc_r[...]
    num = a[0, :, :, 0:_D] + a[1, :, :, 0:_D]
    s = a[0, :, :, _D:_D + 1] + a[1, :, :, _D:_D + 1]
    o = num / (s + 1e-9)
    x2 = jnp.where(o > 0, o, jnp.exp(o) - 1.0)
    blk = x2.shape[1]
    w = w_r[...]
    h2 = jnp.zeros((blk, _D), jnp.float32)
    for hd in range(_H1):
        h2 = h2 + jnp.dot(x2[hd], w[hd * _D:(hd + 1) * _D, :],
                          preferred_element_type=jnp.float32)
    pad = jnp.concatenate(
        [jnp.ones((blk, 1), jnp.float32),
         jnp.zeros((blk, _DA - _D - 1), jnp.float32)], axis=1)
    haug_r[0, :, 0:_D] = h2
    haug_r[0, :, _D:_DA] = pad
    el = jnp.dot(h2, al_r[...][0], preferred_element_type=jnp.float32)
    er = jnp.dot(h2, ar_r[...][0], preferred_element_type=jnp.float32)
    el_r[0] = el.reshape(-1, 128)
    er_r[0] = er.reshape(-1, 128)
    bm = jnp.broadcast_to(jnp.max(el), (1, 128))

    @pl.when(pl.program_id(0) == 0)
    def _():
        gm_r[...] = bm

    @pl.when(pl.program_id(0) > 0)
    def _():
        gm_r[...] = jnp.maximum(gm_r[...], bm)


def _prep2(acc1, W2p, al2p, ar2p):
    blk = 2048
    grid = (_NP // blk,)
    return pl.pallas_call(
        _prep2_body,
        grid=grid,
        in_specs=[
            pl.BlockSpec((_NCORE, _H1, blk, _DA), lambda i: (0, 0, i, 0)),
            pl.BlockSpec((_H1 * _D, _D), lambda i: (0, 0)),
            pl.BlockSpec((1, _D), lambda i: (0, 0)),
            pl.BlockSpec((1, _D), lambda i: (0, 0)),
        ],
        out_specs=[
            pl.BlockSpec((1, blk, _DA), lambda i: (0, i, 0)),
            pl.BlockSpec((1, blk // 128, 128), lambda i: (0, i, 0)),
            pl.BlockSpec((1, blk // 128, 128), lambda i: (0, i, 0)),
            pl.BlockSpec((1, 128), lambda i: (0, 0)),
        ],
        out_shape=[
            jax.ShapeDtypeStruct((1, _NP, _DA), jnp.float32),
            jax.ShapeDtypeStruct((1, _NP // 128, 128), jnp.float32),
            jax.ShapeDtypeStruct((1, _NP // 128, 128), jnp.float32),
            jax.ShapeDtypeStruct((1, 128), jnp.float32),
        ],
    )(acc1, W2p, al2p, ar2p)


def _final_body(acc_r, out_r):
    a = acc_r[...]
    num = a[0, 0, :, 0:_NC] + a[1, 0, :, 0:_NC]
    s = a[0, 0, :, _D:_D + 1] + a[1, 0, :, _D:_D + 1]
    out_r[...] = num / (s + 1e-9)


def _final(acc2):
    blk = 2000
    grid = (_N // blk,)
    return pl.pallas_call(
        _final_body,
        grid=grid,
        in_specs=[pl.BlockSpec((_NCORE, 1, blk, _DA), lambda i: (0, 0, i, 0))],
        out_specs=[pl.BlockSpec((blk, _NC), lambda i: (i, 0))],
        out_shape=[jax.ShapeDtypeStruct((_N, _NC), jnp.float32)],
    )(acc2)[0]


def _sc_edge(H):
    mesh = plsc.VectorSubcoreMesh(core_axis_name="c", subcore_axis_name="s")
    NZ = _N // _B

    def body(haug, elT, erT, gmax, esrc, edst, acc_out,
             ebsrc, ebdst, el_v, er_v, gm_v,
             exbuf, srcbuf, dstbuf, gbuf, acc_sh):
        c = lax.axis_index("c")
        s = lax.axis_index("s")
        wid = s * _NCORE + c
        pltpu.sync_copy(gmax, gm_v)

        def head(hd, _):
            pltpu.sync_copy(elT.at[hd], el_v)
            pltpu.sync_copy(erT.at[hd], er_v)
            gmv = gm_v[hd, pl.ds(0, 16)]

            def zrow(i, _):
                for kk in range(_DA // 16):
                    gbuf[i, pl.ds(kk * 16, 16)] = jnp.zeros((16,), jnp.float32)
                return 0
            lax.fori_loop(0, _B, zrow, 0)
            for z in range(10):
                zi = s * 10 + z

                @pl.when(zi < NZ)
                def _():
                    pltpu.sync_copy(gbuf, acc_sh.at[pl.ds(zi * _B, _B)])

            @pl.when(s == 0)
            def _():
                pltpu.sync_copy(gbuf.at[pl.ds(0, 16)],
                                acc_sh.at[pl.ds(NZ * _B, 16)])
            plsc.subcore_barrier()

            def chunk(ci, _):
                row0 = wid * _RW + ci * _CR
                pltpu.sync_copy(esrc.at[pl.ds(row0, _CR)], ebsrc)
                pltpu.sync_copy(edst.at[pl.ds(row0, _CR)], ebdst)
                for bi in range(_NBC):
                    r0 = bi * _B // 128
                    c0 = (bi * _B) % 128
                    for j in range(_B // 16):
                        rr = r0 + (c0 + j * 16) // 128
                        cc = (c0 + j * 16) % 128
                        src_v = ebsrc[rr, pl.ds(cc, 16)]
                        dst_v = ebdst[rr, pl.ds(cc, 16)]
                        a = plsc.load_gather(
                            el_v, [src_v >> 7, src_v & 127])
                        r = plsc.load_gather(
                            er_v, [dst_v >> 7, dst_v & 127])
                        z = a + r
                        e = jnp.where(z > 0, z, _NEG * z)
                        mb = gmv + r
                        m = jnp.where(mb > 0, mb, _NEG * mb)
                        ei = ((row0 + rr) * 128 + cc
                              + lax.iota(jnp.int32, 16))
                        ex = jnp.where(ei < _E, jnp.exp(e - m), 0.0)
                        exbuf[pl.ds(j * 16, 16)] = ex
                        srcbuf[pl.ds(j * 16, 16)] = src_v + hd * _NP
                        dstbuf[pl.ds(j * 16, 16)] = dst_v
                    pltpu.sync_copy(haug.at[srcbuf], gbuf)

                    def scale(e_i, _):
                        exv = plsc.load_gather(
                            exbuf, [jnp.zeros((16,), jnp.int32) + e_i])
                        for kk in range(5):
                            gbuf[e_i, pl.ds(kk * 16, 16)] = (
                                gbuf[e_i, pl.ds(kk * 16, 16)] * exv)
                        return 0
                    lax.fori_loop(0, _B, scale, 0)
                    pltpu.sync_copy(gbuf, acc_sh.at[dstbuf], add=True)
                return 0
            lax.fori_loop(0, _NCH, chunk, 0)
            plsc.subcore_barrier()

            @pl.when(s < 10)
            def _():
                pltpu.sync_copy(acc_sh.at[pl.ds(s * 1000, 1000)],
                                acc_out.at[c, hd, pl.ds(s * 1000, 1000)])
            plsc.subcore_barrier()
            return 0
        lax.fori_loop(0, H, head, 0)

    kern = pl.kernel(
        body,
        out_type=jax.ShapeDtypeStruct((_NCORE, H, _N, _DA), jnp.float32),
        mesh=mesh,
        compiler_params=pltpu.CompilerParams(needs_layout_passes=False),
        scratch_types=[
            pltpu.VMEM((_CR, 128), jnp.int32),
            pltpu.VMEM((_CR, 128), jnp.int32),
            pltpu.VMEM((_NP // 128, 128), jnp.float32),
            pltpu.VMEM((_NP // 128, 128), jnp.float32),
            pltpu.VMEM((H, 128), jnp.float32),
            pltpu.VMEM((_B,), jnp.float32),
            pltpu.VMEM((_B,), jnp.int32),
            pltpu.VMEM((_B,), jnp.int32),
            pltpu.VMEM((_B, _DA), jnp.float32),
            pltpu.MemorySpace.VMEM_SHARED((_N + 16, _DA), jnp.float32),
        ],
    )
    return kern


_sc_edge8 = _sc_edge(_H1)
_sc_edge1 = _sc_edge(1)


def kernel(x, edge_index, W1, al1, ar1, W2, al2, ar2):
    epad = jnp.pad(edge_index, ((0, 0), (0, _ER * 128 - _E)))
    esrc = epad[0].reshape(_ER, 128)
    edst = epad[1].reshape(_ER, 128)

    xp = jnp.pad(x, ((0, _NP - _N), (0, 0)))
    haug1, el1, er1, gm1 = _prep1(xp, W1, al1, ar1)
    acc1 = _sc_edge8(haug1.reshape(_H1 * _NP, _DA), el1, er1, gm1, esrc, edst)

    W2p = jnp.zeros((_H1 * _D, _D), jnp.float32).at[:, :_NC].set(W2)
    al2p = jnp.zeros((1, _D), jnp.float32).at[:, :_NC].set(al2)
    ar2p = jnp.zeros((1, _D), jnp.float32).at[:, :_NC].set(ar2)

    acc1p = jnp.pad(acc1, ((0, 0), (0, 0), (0, _NP - _N), (0, 0)))
    haug2, el2, er2, gm2 = _prep2(acc1p, W2p, al2p, ar2p)
    acc2 = _sc_edge1(haug2.reshape(_NP, _DA), el2, er2, gm2, esrc, edst)

    return _final(acc2)


# speedup vs baseline: 9.1161x; 1.0603x over previous
import jax
import jax.numpy as jnp
from jax import lax
from jax.experimental import pallas as pl
from jax.experimental.pallas import tpu as pltpu
from jax.experimental.pallas import tpu_sc as plsc

_N = 10000
_E = 320000
_IN = 128
_D = 64
_H1 = 8
_NC = 41
_NEG = 0.2
_DA = 128
_NCORE = 2
_NSUB = 16
_NW = _NCORE * _NSUB
_NP = 10240
_ER = 2560
_RW = _ER // _NW
_CR = 8
_NCH = _RW // _CR
_B = 128
_NBC = _CR * 128 // _B


def _prep1_body(x_r, w_r, al_r, ar_r, haug_r, el_r, er_r, gm_r):
    h = jnp.dot(x_r[...], w_r[...], preferred_element_type=jnp.float32)
    blk = h.shape[0]
    pad = jnp.concatenate(
        [jnp.ones((blk, 1), jnp.float32),
         jnp.zeros((blk, _DA - _D - 1), jnp.float32)], axis=1)
    al = al_r[...]
    ar = ar_r[...]
    mx = []
    for hd in range(_H1):
        hh = h[:, hd * _D:(hd + 1) * _D]
        haug_r[hd, :, 0:_D] = hh
        haug_r[hd, :, _D:_DA] = pad
        el = jnp.dot(hh, al[hd], preferred_element_type=jnp.float32)
        er = jnp.dot(hh, ar[hd], preferred_element_type=jnp.float32)
        el_r[hd] = el.reshape(-1, 128)
        er_r[hd] = er.reshape(-1, 128)
        mx.append(jnp.max(el))
    bm = jnp.broadcast_to(jnp.stack(mx)[:, None], (_H1, 128))

    @pl.when(pl.program_id(0) == 0)
    def _():
        gm_r[...] = bm

    @pl.when(pl.program_id(0) > 0)
    def _():
        gm_r[...] = jnp.maximum(gm_r[...], bm)


def _prep1(x, W1, al1, ar1):
    blk = 2048
    grid = (_NP // blk,)
    return pl.pallas_call(
        _prep1_body,
        grid=grid,
        in_specs=[
            pl.BlockSpec((blk, _IN), lambda i: (i, 0)),
            pl.BlockSpec((_IN, _H1 * _D), lambda i: (0, 0)),
            pl.BlockSpec((_H1, _D), lambda i: (0, 0)),
            pl.BlockSpec((_H1, _D), lambda i: (0, 0)),
        ],
        out_specs=[
            pl.BlockSpec((_H1, blk, _DA), lambda i: (0, i, 0)),
            pl.BlockSpec((_H1, blk // 128, 128), lambda i: (0, i, 0)),
            pl.BlockSpec((_H1, blk // 128, 128), lambda i: (0, i, 0)),
            pl.BlockSpec((_H1, 128), lambda i: (0, 0)),
        ],
        out_shape=[
            jax.ShapeDtypeStruct((_H1, _NP, _DA), jnp.float32),
            jax.ShapeDtypeStruct((_H1, _NP // 128, 128), jnp.float32),
            jax.ShapeDtypeStruct((_H1, _NP // 128, 128), jnp.float32),
            jax.ShapeDtypeStruct((_H1, 128), jnp.float32),
        ],
    )(x, W1, al1, ar1)


def _prep2_body(acc_r, w_r, al_r, ar_r, haug_r, el_r, er_r, gm_r):
    a = acc_r[...]
    num = a[0, :, :, 0:_D] + a[1, :, :, 0:_D]
    s = a[0, :, :, _D:_D + 1] + a[1, :, :, _D:_D + 1]
    o = num / (s + 1e-9)
    x2 = jnp.where(o > 0, o, jnp.exp(o) - 1.0)
    blk = x2.shape[1]
    w = w_r[...]
    h2 = jnp.zeros((blk, _D), jnp.float32)
    for hd in range(_H1):
        h2 = h2 + jnp.dot(x2[hd], w[hd * _D:(hd + 1) * _D, :],
                          preferred_element_type=jnp.float32)
    pad = jnp.concatenate(
        [jnp.ones((blk, 1), jnp.float32),
         jnp.zeros((blk, _DA - _D - 1), jnp.float32)], axis=1)
    haug_r[0, :, 0:_D] = h2
    haug_r[0, :, _D:_DA] = pad
    el = jnp.dot(h2, al_r[...][0], preferred_element_type=jnp.float32)
    er = jnp.dot(h2, ar_r[...][0], preferred_element_type=jnp.float32)
    el_r[0] = el.reshape(-1, 128)
    er_r[0] = er.reshape(-1, 128)
    bm = jnp.broadcast_to(jnp.max(el), (1, 128))

    @pl.when(pl.program_id(0) == 0)
    def _():
        gm_r[...] = bm

    @pl.when(pl.program_id(0) > 0)
    def _():
        gm_r[...] = jnp.maximum(gm_r[...], bm)


def _prep2(acc1, W2p, al2p, ar2p):
    blk = 2048
    grid = (_NP // blk,)
    return pl.pallas_call(
        _prep2_body,
        grid=grid,
        in_specs=[
            pl.BlockSpec((_NCORE, _H1, blk, _DA), lambda i: (0, 0, i, 0)),
            pl.BlockSpec((_H1 * _D, _D), lambda i: (0, 0)),
            pl.BlockSpec((1, _D), lambda i: (0, 0)),
            pl.BlockSpec((1, _D), lambda i: (0, 0)),
        ],
        out_specs=[
            pl.BlockSpec((1, blk, _DA), lambda i: (0, i, 0)),
            pl.BlockSpec((1, blk // 128, 128), lambda i: (0, i, 0)),
            pl.BlockSpec((1, blk // 128, 128), lambda i: (0, i, 0)),
            pl.BlockSpec((1, 128), lambda i: (0, 0)),
        ],
        out_shape=[
            jax.ShapeDtypeStruct((1, _NP, _DA), jnp.float32),
            jax.ShapeDtypeStruct((1, _NP // 128, 128), jnp.float32),
            jax.ShapeDtypeStruct((1, _NP // 128, 128), jnp.float32),
            jax.ShapeDtypeStruct((1, 128), jnp.float32),
        ],
    )(acc1, W2p, al2p, ar2p)


def _final_body(acc_r, out_r):
    a = acc_r[...]
    num = a[0, 0, :, 0:_NC] + a[1, 0, :, 0:_NC]
    s = a[0, 0, :, _D:_D + 1] + a[1, 0, :, _D:_D + 1]
    out_r[...] = num / (s + 1e-9)


def _final(acc2):
    blk = 2000
    grid = (_N // blk,)
    return pl.pallas_call(
        _final_body,
        grid=grid,
        in_specs=[pl.BlockSpec((_NCORE, 1, blk, _DA), lambda i: (0, 0, i, 0))],
        out_specs=[pl.BlockSpec((blk, _NC), lambda i: (i, 0))],
        out_shape=[jax.ShapeDtypeStruct((_N, _NC), jnp.float32)],
    )(acc2)[0]


def _sc_edge(H):
    mesh = plsc.VectorSubcoreMesh(core_axis_name="c", subcore_axis_name="s")
    NZ = _N // _B

    def body(haug, elT, erT, gmax, esrc, edst, acc_out,
             ebsrc, ebdst, el_v, er_v, gm_v,
             exbuf, srcbuf, dstbuf, gbuf, acc_sh):
        c = lax.axis_index("c")
        s = lax.axis_index("s")
        wid = s * _NCORE + c
        pltpu.sync_copy(gmax, gm_v)

        def head(hd, _):
            pltpu.sync_copy(elT.at[hd], el_v)
            pltpu.sync_copy(erT.at[hd], er_v)
            gmv = gm_v[hd, pl.ds(0, 16)]

            def zrow(i, _):
                for kk in range(_DA // 16):
                    gbuf[i, pl.ds(kk * 16, 16)] = jnp.zeros((16,), jnp.float32)
                return 0
            lax.fori_loop(0, _B, zrow, 0)
            for z in range(5):
                zi = s * 5 + z

                @pl.when(zi < NZ)
                def _():
                    pltpu.sync_copy(gbuf, acc_sh.at[pl.ds(zi * _B, _B)])

            @pl.when(s == 0)
            def _():
                pltpu.sync_copy(gbuf.at[pl.ds(0, 16)],
                                acc_sh.at[pl.ds(NZ * _B, 16)])
            plsc.subcore_barrier()

            def chunk(ci, _):
                row0 = wid * _RW + ci * _CR
                pltpu.sync_copy(esrc.at[pl.ds(row0, _CR)], ebsrc)
                pltpu.sync_copy(edst.at[pl.ds(row0, _CR)], ebdst)
                for bi in range(_NBC):
                    r0 = bi * _B // 128
                    c0 = (bi * _B) % 128
                    for j in range(_B // 16):
                        rr = r0 + (c0 + j * 16) // 128
                        cc = (c0 + j * 16) % 128
                        src_v = ebsrc[rr, pl.ds(cc, 16)]
                        dst_v = ebdst[rr, pl.ds(cc, 16)]
                        a = plsc.load_gather(
                            el_v, [src_v >> 7, src_v & 127])
                        r = plsc.load_gather(
                            er_v, [dst_v >> 7, dst_v & 127])
                        z = a + r
                        e = jnp.where(z > 0, z, _NEG * z)
                        mb = gmv + r
                        m = jnp.where(mb > 0, mb, _NEG * mb)
                        ei = ((row0 + rr) * 128 + cc
                              + lax.iota(jnp.int32, 16))
                        ex = jnp.where(ei < _E, jnp.exp(e - m), 0.0)
                        exbuf[pl.ds(j * 16, 16)] = ex
                        srcbuf[pl.ds(j * 16, 16)] = src_v + hd * _NP
                        dstbuf[pl.ds(j * 16, 16)] = dst_v
                    pltpu.sync_copy(haug.at[srcbuf], gbuf)

                    def scale(e_i, _):
                        exv = plsc.load_gather(
                            exbuf, [jnp.zeros((16,), jnp.int32) + e_i])
                        for kk in range(5):
                            gbuf[e_i, pl.ds(kk * 16, 16)] = (
                                gbuf[e_i, pl.ds(kk * 16, 16)] * exv)
                        return 0
                    lax.fori_loop(0, _B, scale, 0)
                    pltpu.sync_copy(gbuf, acc_sh.at[dstbuf], add=True)
                return 0
            lax.fori_loop(0, _NCH, chunk, 0)
            plsc.subcore_barrier()

            @pl.when(s < 10)
            def _():
                pltpu.sync_copy(acc_sh.at[pl.ds(s * 1000, 1000)],
                                acc_out.at[c, hd, pl.ds(s * 1000, 1000)])
            plsc.subcore_barrier()
            return 0
        lax.fori_loop(0, H, head, 0)

    kern = pl.kernel(
        body,
        out_type=jax.ShapeDtypeStruct((_NCORE, H, _N, _DA), jnp.float32),
        mesh=mesh,
        compiler_params=pltpu.CompilerParams(needs_layout_passes=False),
        scratch_types=[
            pltpu.VMEM((_CR, 128), jnp.int32),
            pltpu.VMEM((_CR, 128), jnp.int32),
            pltpu.VMEM((_NP // 128, 128), jnp.float32),
            pltpu.VMEM((_NP // 128, 128), jnp.float32),
            pltpu.VMEM((H, 128), jnp.float32),
            pltpu.VMEM((_B,), jnp.float32),
            pltpu.VMEM((_B,), jnp.int32),
            pltpu.VMEM((_B,), jnp.int32),
            pltpu.VMEM((_B, _DA), jnp.float32),
            pltpu.MemorySpace.VMEM_SHARED((_N + 16, _DA), jnp.float32),
        ],
    )
    return kern


_sc_edge8 = _sc_edge(_H1)
_sc_edge1 = _sc_edge(1)


def kernel(x, edge_index, W1, al1, ar1, W2, al2, ar2):
    epad = jnp.pad(edge_index, ((0, 0), (0, _ER * 128 - _E)))
    esrc = epad[0].reshape(_ER, 128)
    edst = epad[1].reshape(_ER, 128)

    xp = jnp.pad(x, ((0, _NP - _N), (0, 0)))
    haug1, el1, er1, gm1 = _prep1(xp, W1, al1, ar1)
    acc1 = _sc_edge8(haug1.reshape(_H1 * _NP, _DA), el1, er1, gm1, esrc, edst)

    W2p = jnp.zeros((_H1 * _D, _D), jnp.float32).at[:, :_NC].set(W2)
    al2p = jnp.zeros((1, _D), jnp.float32).at[:, :_NC].set(al2)
    ar2p = jnp.zeros((1, _D), jnp.float32).at[:, :_NC].set(ar2)

    acc1p = jnp.pad(acc1, ((0, 0), (0, 0), (0, _NP - _N), (0, 0)))
    haug2, el2, er2, gm2 = _prep2(acc1p, W2p, al2p, ar2p)
    acc2 = _sc_edge1(haug2.reshape(_NP, _DA), el2, er2, gm2, esrc, edst)

    return _final(acc2)
